# baseline (device time: 11991 ns/iter reference)
import jax
import jax.numpy as jnp
from jax import lax
from jax.experimental import pallas as pl
from jax.experimental.pallas import tpu as pltpu

N_DEV = 32


def kernel(x):
    m, n = x.shape

    def body(x_ref, out_ref, halo_ref, send_sems, recv_sems):
        my = lax.axis_index("i")
        left = lax.rem(my + N_DEV - 1, N_DEV)
        right = lax.rem(my + 1, N_DEV)

        barrier_sem = pltpu.get_barrier_semaphore()
        for nbr in (left, right):
            pl.semaphore_signal(
                barrier_sem, inc=1,
                device_id=(nbr,), device_id_type=pl.DeviceIdType.MESH,
            )
        pl.semaphore_wait(barrier_sem, 2)

        rdma_r = pltpu.make_async_remote_copy(
            src_ref=x_ref.at[pl.ds(m - 1, 1), :],
            dst_ref=halo_ref.at[pl.ds(0, 1), :],
            send_sem=send_sems.at[0],
            recv_sem=recv_sems.at[0],
            device_id=(right,),
            device_id_type=pl.DeviceIdType.MESH,
        )
        rdma_l = pltpu.make_async_remote_copy(
            src_ref=x_ref.at[pl.ds(0, 1), :],
            dst_ref=halo_ref.at[pl.ds(1, 1), :],
            send_sem=send_sems.at[1],
            recv_sem=recv_sems.at[1],
            device_id=(left,),
            device_id_type=pl.DeviceIdType.MESH,
        )
        rdma_r.start()
        rdma_l.start()

        xm = x_ref[:, :]
        up = jnp.concatenate([xm[0:1, :], xm[: m - 1, :]], axis=0)
        dn = jnp.concatenate([xm[1:, :], xm[m - 1 :, :]], axis=0)
        out_ref[:, :] = 0.25 * up + 0.5 * xm + 0.25 * dn

        rdma_r.wait()
        rdma_l.wait()

        x0 = x_ref[pl.ds(0, 1), :]
        top = 0.25 * halo_ref[pl.ds(0, 1), :] + 0.5 * x0 + 0.25 * x_ref[pl.ds(1, 1), :]
        out_ref[pl.ds(0, 1), :] = jnp.where(my == 0, x0, top)

        xl = x_ref[pl.ds(m - 1, 1), :]
        bot = 0.25 * x_ref[pl.ds(m - 2, 1), :] + 0.5 * xl + 0.25 * halo_ref[pl.ds(1, 1), :]
        out_ref[pl.ds(m - 1, 1), :] = jnp.where(my == N_DEV - 1, xl, bot)

    return pl.pallas_call(
        body,
        out_shape=jax.ShapeDtypeStruct((m, n), x.dtype),
        in_specs=[pl.BlockSpec(memory_space=pltpu.VMEM)],
        out_specs=pl.BlockSpec(memory_space=pltpu.VMEM),
        scratch_shapes=[
            pltpu.VMEM((2, n), x.dtype),
            pltpu.SemaphoreType.DMA((2,)),
            pltpu.SemaphoreType.DMA((2,)),
        ],
        compiler_params=pltpu.CompilerParams(collective_id=0),
    )(x)


# device time: 6769 ns/iter; 1.7715x vs baseline; 1.7715x over previous
import jax
import jax.numpy as jnp
from jax import lax
from jax.experimental import pallas as pl
from jax.experimental.pallas import tpu as pltpu

N_DEV = 32


def kernel(x):
    m, n = x.shape

    def body(x_ref, out_ref, halo_ref, send_sems, recv_sems):
        my = lax.axis_index("i")
        left = lax.rem(my + N_DEV - 1, N_DEV)
        right = lax.rem(my + 1, N_DEV)
        has_left = my > 0
        has_right = my < N_DEV - 1

        barrier_sem = pltpu.get_barrier_semaphore()

        @pl.when(has_left)
        def _():
            pl.semaphore_signal(
                barrier_sem, inc=1,
                device_id=(left,), device_id_type=pl.DeviceIdType.MESH,
            )
            pl.semaphore_wait(barrier_sem, 1)

        @pl.when(has_right)
        def _():
            pl.semaphore_signal(
                barrier_sem, inc=1,
                device_id=(right,), device_id_type=pl.DeviceIdType.MESH,
            )
            pl.semaphore_wait(barrier_sem, 1)

        rdma_r = pltpu.make_async_remote_copy(
            src_ref=x_ref.at[pl.ds(m - 1, 1), :],
            dst_ref=halo_ref.at[pl.ds(0, 1), :],
            send_sem=send_sems.at[0],
            recv_sem=recv_sems.at[0],
            device_id=(right,),
            device_id_type=pl.DeviceIdType.MESH,
        )
        rdma_l = pltpu.make_async_remote_copy(
            src_ref=x_ref.at[pl.ds(0, 1), :],
            dst_ref=halo_ref.at[pl.ds(1, 1), :],
            send_sem=send_sems.at[1],
            recv_sem=recv_sems.at[1],
            device_id=(left,),
            device_id_type=pl.DeviceIdType.MESH,
        )

        @pl.when(has_right)
        def _():
            rdma_r.start()

        @pl.when(has_left)
        def _():
            rdma_l.start()

        xm = x_ref[:, :]
        up = jnp.concatenate([xm[0:1, :], xm[: m - 1, :]], axis=0)
        dn = jnp.concatenate([xm[1:, :], xm[m - 1 :, :]], axis=0)
        out_ref[:, :] = 0.25 * up + 0.5 * xm + 0.25 * dn

        @pl.when(has_left)
        def _():
            rdma_r.wait_recv()
            rdma_l.wait_send()
            out_ref[pl.ds(0, 1), :] = (
                0.25 * halo_ref[pl.ds(0, 1), :]
                + 0.5 * x_ref[pl.ds(0, 1), :]
                + 0.25 * x_ref[pl.ds(1, 1), :]
            )

        @pl.when(jnp.logical_not(has_left))
        def _():
            out_ref[pl.ds(0, 1), :] = x_ref[pl.ds(0, 1), :]

        @pl.when(has_right)
        def _():
            rdma_l.wait_recv()
            rdma_r.wait_send()
            out_ref[pl.ds(m - 1, 1), :] = (
                0.25 * x_ref[pl.ds(m - 2, 1), :]
                + 0.5 * x_ref[pl.ds(m - 1, 1), :]
                + 0.25 * halo_ref[pl.ds(1, 1), :]
            )

        @pl.when(jnp.logical_not(has_right))
        def _():
            out_ref[pl.ds(m - 1, 1), :] = x_ref[pl.ds(m - 1, 1), :]

    return pl.pallas_call(
        body,
        out_shape=jax.ShapeDtypeStruct((m, n), x.dtype),
        in_specs=[pl.BlockSpec(memory_space=pltpu.VMEM)],
        out_specs=pl.BlockSpec(memory_space=pltpu.VMEM),
        scratch_shapes=[
            pltpu.VMEM((2, n), x.dtype),
            pltpu.SemaphoreType.DMA((2,)),
            pltpu.SemaphoreType.DMA((2,)),
        ],
        compiler_params=pltpu.CompilerParams(collective_id=0),
    )(x)
